# baseline (speedup 1.0000x reference)
import functools

import jax
import jax.numpy as jnp
from jax import lax
from jax.experimental import pallas as pl
from jax.experimental.pallas import tpu as pltpu
from jax.experimental.pallas import tpu_sc as plsc

N = 50000
E = 1600000
NC = 2
NS = 16
NW = NC * NS
EPW = E // NW
NPAD = 50176
NBLK = 12544
BPW = NBLK // NW
NSLICE = NPAD // NS
ZROWS = 64

_MESH = plsc.VectorSubcoreMesh(core_axis_name="c", subcore_axis_name="s")

DEG_BLKS = 8


@functools.partial(
    pl.kernel,
    out_type=jax.ShapeDtypeStruct((NC * NPAD,), jnp.float32),
    mesh=_MESH,
    scratch_types=[
        pltpu.VMEM((DEG_BLKS, 256), jnp.int32),
        pltpu.VMEM((DEG_BLKS * 128,), jnp.int32),
        pltpu.VMEM((DEG_BLKS * 128,), jnp.float32),
        pltpu.VMEM((NSLICE,), jnp.float32),
        pltpu.VMEM_SHARED((NPAD,), jnp.float32),
        pltpu.SemaphoreType.DMA,
    ],
    compiler_params=pltpu.CompilerParams(
        needs_layout_passes=False, use_tc_tiling_on_sc=False),
)
def _deg_kernel(eint_hbm, ew_hbm, out_hbm, ebuf, cidx, ew_v, zbuf, deg_sh,
                sem):
    c = lax.axis_index("c")
    s = lax.axis_index("s")
    blk0 = (c * NS + s) * BPW

    def zfill(i, carry):
        zbuf[pl.ds(i * 16, 16)] = jnp.zeros((16,), jnp.float32)
        return carry

    lax.fori_loop(0, NSLICE // 16, zfill, 0)
    pltpu.sync_copy(zbuf, deg_sh.at[pl.ds(s * NSLICE, NSLICE)])
    plsc.subcore_barrier()

    def body(k, carry):
        b0 = blk0 + k * DEG_BLKS
        pltpu.sync_copy(eint_hbm.at[pl.ds(b0, DEG_BLKS)], ebuf)
        pltpu.sync_copy(ew_hbm.at[pl.ds(b0 * 128, DEG_BLKS * 128)], ew_v)
        for t in range(DEG_BLKS):
            for q in range(8):
                cidx[pl.ds(t * 128 + q * 16, 16)] = (
                    ebuf[t, pl.ds(128 + q * 16, 16)])
        pltpu.sync_copy(ew_v, deg_sh.at[cidx], add=True)
        return carry

    lax.fori_loop(0, BPW // DEG_BLKS, body, 0)
    plsc.subcore_barrier()
    pltpu.sync_copy(deg_sh.at[pl.ds(s * NSLICE, NSLICE)], zbuf)
    pltpu.sync_copy(zbuf, out_hbm.at[pl.ds(c * NPAD + s * NSLICE, NSLICE)])


def _make_msg_kernel(D, BLKS):
    CH = BLKS * 128
    nchunks = BPW // BLKS

    @functools.partial(
        pl.kernel,
        out_type=jax.ShapeDtypeStruct((NC, NPAD, D), jnp.float32),
        mesh=_MESH,
        scratch_types=[
            pltpu.VMEM((BLKS, 256), jnp.int32),
            pltpu.VMEM((BLKS, 256), jnp.int32),
            pltpu.VMEM((CH,), jnp.int32),
            pltpu.VMEM((CH,), jnp.int32),
            pltpu.VMEM((CH,), jnp.int32),
            pltpu.VMEM((CH,), jnp.int32),
            pltpu.VMEM((CH,), jnp.float32),
            pltpu.VMEM((CH,), jnp.float32),
            pltpu.VMEM((CH, D), jnp.float32),
            pltpu.VMEM((CH, D), jnp.float32),
            pltpu.VMEM((ZROWS, D), jnp.float32),
            pltpu.VMEM_SHARED((NPAD, D), jnp.float32),
            pltpu.SemaphoreType.DMA,
            pltpu.SemaphoreType.DMA,
        ],
        compiler_params=pltpu.CompilerParams(
            needs_layout_passes=False, use_tc_tiling_on_sc=False),
    )
    def msg_kernel(eint_hbm, ew_hbm, y_hbm, out_hbm,
                   eb0, eb1, ri0, ri1, ci0, ci1, ew0, ew1, g0, g1,
                   zbuf, acc_sh, sem0, sem1):
        c = lax.axis_index("c")
        s = lax.axis_index("s")
        blk0 = (c * NS + s) * BPW
        ebufs = (eb0, eb1)
        ridx = (ri0, ri1)
        cidx = (ci0, ci1)
        ewb = (ew0, ew1)
        gbufs = (g0, g1)
        sems = (sem0, sem1)

        def zfill(i, carry):
            for h in range(D // 16):
                zbuf[i, pl.ds(h * 16, 16)] = jnp.zeros((16,), jnp.float32)
            return carry

        lax.fori_loop(0, ZROWS, zfill, 0)
        for j in range(NSLICE // ZROWS):
            pltpu.sync_copy(
                zbuf, acc_sh.at[pl.ds(s * NSLICE + j * ZROWS, ZROWS)])
        plsc.subcore_barrier()

        def fetch(k, b):
            b0 = blk0 + k * BLKS
            pltpu.sync_copy(eint_hbm.at[pl.ds(b0, BLKS)], ebufs[b])
            pltpu.sync_copy(ew_hbm.at[pl.ds(b0 * 128, CH)], ewb[b])
            for t in range(BLKS):
                for q in range(8):
                    sl = pl.ds(t * 128 + q * 16, 16)
                    ridx[b][sl] = ebufs[b][t, pl.ds(q * 16, 16)]
                    cidx[b][sl] = ebufs[b][t, pl.ds(128 + q * 16, 16)]
            pltpu.async_copy(y_hbm.at[ridx[b]], gbufs[b], sems[b])

        fetch(0, 0)

        def step(k, b):
            @pl.when(k + 1 < nchunks)
            def _():
                fetch(k + 1, 1 - b)

            g_v = gbufs[b]
            ew_v = ewb[b]
            pltpu.make_async_copy(y_hbm.at[ridx[b]], g_v, sems[b]).wait()

            lanes = [jnp.full((16, 1), p, jnp.int32) for p in range(16)]
            gdn = lax.GatherDimensionNumbers(
                offset_dims=(), collapsed_slice_dims=(0,),
                start_index_map=(0,))

            @plsc.parallel_loop(0, CH // 16, unroll=2)
            def scale_group(j):
                ew16 = ew_v[pl.ds(j * 16, 16)]
                for p in range(16):
                    w = lax.gather(
                        ew16, lanes[p], gdn, (1,),
                        mode=lax.GatherScatterMode.PROMISE_IN_BOUNDS)
                    e = j * 16 + p
                    for h in range(D // 16):
                        g_v[e, pl.ds(h * 16, 16)] = (
                            g_v[e, pl.ds(h * 16, 16)] * w)

            pltpu.sync_copy(g_v, acc_sh.at[cidx[b]], add=True)

        def pair_body(i, carry):
            step(2 * i, 0)
            step(2 * i + 1, 1)
            return carry

        lax.fori_loop(0, nchunks // 2, pair_body, 0)
        if nchunks % 2:
            step(nchunks - 1, 0)
        plsc.subcore_barrier()
        for j in range(NSLICE // ZROWS):
            r0 = s * NSLICE + j * ZROWS
            pltpu.sync_copy(acc_sh.at[pl.ds(r0, ZROWS)], zbuf)
            pltpu.sync_copy(zbuf, out_hbm.at[c, pl.ds(r0, ZROWS)])

    return msg_kernel


_msg32 = _make_msg_kernel(32, 2)
_msg16 = _make_msg_kernel(16, 2)

_R1 = 1000


def _tcmm_body(xt_ref, w1_ref, xw_ref):
    xw_ref[...] = lax.dot_general(
        xt_ref[...], w1_ref[...], (((0,), (0,)), ((), ())),
        preferred_element_type=jnp.float32)


def _tcmm(xt, W1):
    grid = (N + 1023) // 1024
    return pl.pallas_call(
        _tcmm_body,
        grid=(grid,),
        in_specs=[
            pl.BlockSpec((1000, 1024), lambda i: (0, i)),
            pl.BlockSpec((1000, 32), lambda i: (0, 0)),
        ],
        out_specs=pl.BlockSpec((1024, 32), lambda i: (i, 0)),
        out_shape=jax.ShapeDtypeStruct((N, 32), jnp.float32),
    )(xt, W1)


def _tcpre_body(xw_ref, degp_ref, y_ref, dinv_ref):
    deg = degp_ref[:, 0] + degp_ref[:, 1] + 1.0
    dinv = lax.rsqrt(deg)
    y_ref[...] = xw_ref[...] * dinv[:, None]
    dinv_ref[...] = dinv[:, None]


def _tcpre(xw, degp):
    grid = N // _R1
    return pl.pallas_call(
        _tcpre_body,
        grid=(grid,),
        in_specs=[
            pl.BlockSpec((_R1, 32), lambda i: (i, 0)),
            pl.BlockSpec((_R1, NC), lambda i: (i, 0)),
        ],
        out_specs=[
            pl.BlockSpec((_R1, 32), lambda i: (i, 0)),
            pl.BlockSpec((_R1, 1), lambda i: (i, 0)),
        ],
        out_shape=[
            jax.ShapeDtypeStruct((N, 32), jnp.float32),
            jax.ShapeDtypeStruct((N, 1), jnp.float32),
        ],
    )(xw, degp)


def _tc2_body(sp_ref, y1_ref, dinv_ref, w2_ref, b1_ref, y2_ref):
    dinv = dinv_ref[...]
    h = (sp_ref[0] + sp_ref[1] + y1_ref[...]) * dinv + b1_ref[...]
    h = jnp.maximum(h, 0.0)
    xw2 = jnp.dot(h, w2_ref[...], preferred_element_type=jnp.float32)
    y2_ref[...] = xw2 * dinv


def _tc2(s1p, y1, dinv, W2, b1):
    grid = N // _R1
    return pl.pallas_call(
        _tc2_body,
        grid=(grid,),
        in_specs=[
            pl.BlockSpec((NC, _R1, 32), lambda i: (0, i, 0)),
            pl.BlockSpec((_R1, 32), lambda i: (i, 0)),
            pl.BlockSpec((_R1, 1), lambda i: (i, 0)),
            pl.BlockSpec((32, 16), lambda i: (0, 0)),
            pl.BlockSpec((1, 32), lambda i: (0, 0)),
        ],
        out_specs=pl.BlockSpec((_R1, 16), lambda i: (i, 0)),
        out_shape=jax.ShapeDtypeStruct((N, 16), jnp.float32),
    )(s1p, y1, dinv, W2, b1)


def _tc3_body(sp_ref, y2_ref, dinv_ref, b2_ref, h2_ref):
    h = (sp_ref[0] + sp_ref[1] + y2_ref[...]) * dinv_ref[...] + b2_ref[...]
    h2_ref[...] = jnp.maximum(h, 0.0)


def _tc3(s2p, y2, dinv, b2):
    grid = N // _R1
    return pl.pallas_call(
        _tc3_body,
        grid=(grid,),
        in_specs=[
            pl.BlockSpec((NC, _R1, 16), lambda i: (0, i, 0)),
            pl.BlockSpec((_R1, 16), lambda i: (i, 0)),
            pl.BlockSpec((_R1, 1), lambda i: (i, 0)),
            pl.BlockSpec((1, 16), lambda i: (0, 0)),
        ],
        out_specs=pl.BlockSpec((_R1, 16), lambda i: (i, 0)),
        out_shape=jax.ShapeDtypeStruct((N, 16), jnp.float32),
    )(s2p, y2, dinv, b2)


def _tc4_body(h_ref, wfct_ref, bfc_ref, wfc2t_ref, bfc2_ref, out_ref):
    z = jnp.dot(h_ref[...], wfct_ref[...], preferred_element_type=jnp.float32)
    z = jnp.maximum(z + bfc_ref[...], 0.0)
    o = jnp.dot(z, wfc2t_ref[...], preferred_element_type=jnp.float32)
    out_ref[...] = jax.nn.sigmoid(o + bfc2_ref[...])


def _tc4(hflat, WfcT, bfc, Wfc2T, bfc2):
    return pl.pallas_call(
        _tc4_body,
        out_shape=jax.ShapeDtypeStruct((50, 1), jnp.float32),
    )(hflat, WfcT, bfc, Wfc2T, bfc2)


def kernel(x, edge_index, edge_attr, W1, b1, W2, b2, Wfc, bfc, Wfc2, bfc2):
    ew = edge_attr
    eint = jnp.concatenate(
        [edge_index.reshape(2, E // 128, 128).transpose(1, 0, 2),
         jnp.zeros((NBLK - E // 128, 2, 128), jnp.int32)], 0).reshape(
             NBLK, 256)
    ewp = jnp.concatenate(
        [ew, jnp.zeros((NBLK * 128 - E,), jnp.float32)])

    degp = _deg_kernel(eint, ewp)
    xw1 = _tcmm(x.T, W1)
    y1, dinv = _tcpre(xw1, degp.reshape(NC, NPAD).T)
    s1p = _msg32(eint, ewp, y1)
    y2 = _tc2(s1p, y1, dinv, W2, b1.reshape(1, 32))
    s2p = _msg16(eint, ewp, y2)
    h2 = _tc3(s2p, y2, dinv, b2.reshape(1, 16))
    hflat = h2.reshape(50, 16000)
    return _tc4(hflat, Wfc.T, bfc.reshape(1, 64), Wfc2.T, bfc2.reshape(1, 1))


# speedup vs baseline: 1.1701x; 1.1701x over previous
import functools

import jax
import jax.numpy as jnp
from jax import lax
from jax.experimental import pallas as pl
from jax.experimental.pallas import tpu as pltpu
from jax.experimental.pallas import tpu_sc as plsc

N = 50000
E = 1600000
NC = 2
NS = 16
NW = NC * NS
EPW = E // NW
NPAD = 50176
NSLICE = NPAD // NS
ZROWS = 64

_MESH = plsc.VectorSubcoreMesh(core_axis_name="c", subcore_axis_name="s")

DEG_CH = 400


@functools.partial(
    pl.kernel,
    out_type=jax.ShapeDtypeStruct((NC * NPAD,), jnp.float32),
    mesh=_MESH,
    scratch_types=[
        pltpu.VMEM((3, DEG_CH), jnp.int32),
        pltpu.VMEM((DEG_CH,), jnp.float32),
        pltpu.VMEM((NSLICE,), jnp.float32),
        pltpu.VMEM_SHARED((NPAD,), jnp.float32),
        pltpu.SemaphoreType.DMA,
    ],
    compiler_params=pltpu.CompilerParams(
        needs_layout_passes=False, use_tc_tiling_on_sc=False),
)
def _deg_kernel(edata_hbm, out_hbm, ebuf, ew_v, zbuf, deg_sh, sem):
    c = lax.axis_index("c")
    s = lax.axis_index("s")
    nchunks = EPW // DEG_CH
    kbase = (c * NS + s) * nchunks

    def zfill(i, carry):
        zbuf[pl.ds(i * 16, 16)] = jnp.zeros((16,), jnp.float32)
        return carry

    lax.fori_loop(0, NSLICE // 16, zfill, 0)
    pltpu.sync_copy(zbuf, deg_sh.at[pl.ds(s * NSLICE, NSLICE)])
    plsc.subcore_barrier()

    def body(k, carry):
        pltpu.sync_copy(edata_hbm.at[kbase + k], ebuf)
        for q in range(DEG_CH // 16):
            ew_v[pl.ds(q * 16, 16)] = plsc.bitcast(
                ebuf[2, pl.ds(q * 16, 16)], jnp.float32)
        pltpu.sync_copy(ew_v, deg_sh.at[ebuf.at[1]], add=True)
        return carry

    lax.fori_loop(0, nchunks, body, 0)
    plsc.subcore_barrier()
    pltpu.sync_copy(deg_sh.at[pl.ds(s * NSLICE, NSLICE)], zbuf)
    pltpu.sync_copy(zbuf, out_hbm.at[pl.ds(c * NPAD + s * NSLICE, NSLICE)])


def _make_msg_kernel(D, CH):
    nchunks = EPW // CH

    @functools.partial(
        pl.kernel,
        out_type=jax.ShapeDtypeStruct((NC, NPAD, D), jnp.float32),
        mesh=_MESH,
        scratch_types=[
            pltpu.VMEM((3, CH), jnp.int32),
            pltpu.VMEM((3, CH), jnp.int32),
            pltpu.VMEM((CH, D), jnp.float32),
            pltpu.VMEM((CH, D), jnp.float32),
            pltpu.VMEM((ZROWS, D), jnp.float32),
            pltpu.VMEM_SHARED((NPAD, D), jnp.float32),
            pltpu.SemaphoreType.DMA,
            pltpu.SemaphoreType.DMA,
        ],
        compiler_params=pltpu.CompilerParams(
            needs_layout_passes=False, use_tc_tiling_on_sc=False),
    )
    def msg_kernel(edata_hbm, y_hbm, out_hbm,
                   eb0, eb1, g0, g1, zbuf, acc_sh, sem0, sem1):
        c = lax.axis_index("c")
        s = lax.axis_index("s")
        kbase = (c * NS + s) * nchunks
        ebufs = (eb0, eb1)
        gbufs = (g0, g1)
        sems = (sem0, sem1)

        def zfill(i, carry):
            for h in range(D // 16):
                zbuf[i, pl.ds(h * 16, 16)] = jnp.zeros((16,), jnp.float32)
            return carry

        lax.fori_loop(0, ZROWS, zfill, 0)
        for j in range(NSLICE // ZROWS):
            pltpu.sync_copy(
                zbuf, acc_sh.at[pl.ds(s * NSLICE + j * ZROWS, ZROWS)])
        plsc.subcore_barrier()

        pltpu.sync_copy(edata_hbm.at[kbase], eb0)
        pltpu.async_copy(y_hbm.at[eb0.at[0]], g0, sem0)

        def step(k, b):
            @pl.when(k + 1 < nchunks)
            def _():
                pltpu.sync_copy(edata_hbm.at[kbase + k + 1], ebufs[1 - b])
                pltpu.async_copy(y_hbm.at[ebufs[1 - b].at[0]],
                                 gbufs[1 - b], sems[1 - b])

            g_v = gbufs[b]
            eb = ebufs[b]
            pltpu.make_async_copy(y_hbm.at[eb.at[0]], g_v, sems[b]).wait()

            lanes = [jnp.full((16, 1), p, jnp.int32) for p in range(16)]
            gdn = lax.GatherDimensionNumbers(
                offset_dims=(), collapsed_slice_dims=(0,),
                start_index_map=(0,))

            @plsc.parallel_loop(0, CH // 16, unroll=2)
            def scale_group(j):
                ew16 = plsc.bitcast(eb[2, pl.ds(j * 16, 16)], jnp.float32)
                for p in range(16):
                    w = lax.gather(
                        ew16, lanes[p], gdn, (1,),
                        mode=lax.GatherScatterMode.PROMISE_IN_BOUNDS)
                    e = j * 16 + p
                    for h in range(D // 16):
                        g_v[e, pl.ds(h * 16, 16)] = (
                            g_v[e, pl.ds(h * 16, 16)] * w)

            pltpu.sync_copy(g_v, acc_sh.at[eb.at[1]], add=True)

        def pair_body(i, carry):
            step(2 * i, 0)
            step(2 * i + 1, 1)
            return carry

        lax.fori_loop(0, nchunks // 2, pair_body, 0)
        if nchunks % 2:
            step(nchunks - 1, 0)
        plsc.subcore_barrier()
        for j in range(NSLICE // ZROWS):
            r0 = s * NSLICE + j * ZROWS
            pltpu.sync_copy(acc_sh.at[pl.ds(r0, ZROWS)], zbuf)
            pltpu.sync_copy(zbuf, out_hbm.at[c, pl.ds(r0, ZROWS)])

    return msg_kernel


_msg32 = _make_msg_kernel(32, 400)
_msg16 = _make_msg_kernel(16, 400)

_R1 = 1000


def _tcmm_body(xt_ref, w1_ref, xw_ref):
    xw_ref[...] = lax.dot_general(
        xt_ref[...], w1_ref[...], (((0,), (0,)), ((), ())),
        preferred_element_type=jnp.float32)


def _tcmm(xt, W1):
    grid = (N + 1023) // 1024
    return pl.pallas_call(
        _tcmm_body,
        grid=(grid,),
        in_specs=[
            pl.BlockSpec((1000, 1024), lambda i: (0, i)),
            pl.BlockSpec((1000, 32), lambda i: (0, 0)),
        ],
        out_specs=pl.BlockSpec((1024, 32), lambda i: (i, 0)),
        out_shape=jax.ShapeDtypeStruct((N, 32), jnp.float32),
    )(xt, W1)


def _tcpre_body(xw_ref, degp_ref, y_ref, dinv_ref):
    deg = degp_ref[:, 0] + degp_ref[:, 1] + 1.0
    dinv = lax.rsqrt(deg)
    y_ref[...] = xw_ref[...] * dinv[:, None]
    dinv_ref[...] = dinv[:, None]


def _tcpre(xw, degp):
    grid = N // _R1
    return pl.pallas_call(
        _tcpre_body,
        grid=(grid,),
        in_specs=[
            pl.BlockSpec((_R1, 32), lambda i: (i, 0)),
            pl.BlockSpec((_R1, NC), lambda i: (i, 0)),
        ],
        out_specs=[
            pl.BlockSpec((_R1, 32), lambda i: (i, 0)),
            pl.BlockSpec((_R1, 1), lambda i: (i, 0)),
        ],
        out_shape=[
            jax.ShapeDtypeStruct((N, 32), jnp.float32),
            jax.ShapeDtypeStruct((N, 1), jnp.float32),
        ],
    )(xw, degp)


def _tc2_body(sp_ref, y1_ref, dinv_ref, w2_ref, b1_ref, y2_ref):
    dinv = dinv_ref[...]
    h = (sp_ref[0] + sp_ref[1] + y1_ref[...]) * dinv + b1_ref[...]
    h = jnp.maximum(h, 0.0)
    xw2 = jnp.dot(h, w2_ref[...], preferred_element_type=jnp.float32)
    y2_ref[...] = xw2 * dinv


def _tc2(s1p, y1, dinv, W2, b1):
    grid = N // _R1
    return pl.pallas_call(
        _tc2_body,
        grid=(grid,),
        in_specs=[
            pl.BlockSpec((NC, _R1, 32), lambda i: (0, i, 0)),
            pl.BlockSpec((_R1, 32), lambda i: (i, 0)),
            pl.BlockSpec((_R1, 1), lambda i: (i, 0)),
            pl.BlockSpec((32, 16), lambda i: (0, 0)),
            pl.BlockSpec((1, 32), lambda i: (0, 0)),
        ],
        out_specs=pl.BlockSpec((_R1, 16), lambda i: (i, 0)),
        out_shape=jax.ShapeDtypeStruct((N, 16), jnp.float32),
    )(s1p, y1, dinv, W2, b1)


def _tc3_body(sp_ref, y2_ref, dinv_ref, b2_ref, h2_ref):
    h = (sp_ref[0] + sp_ref[1] + y2_ref[...]) * dinv_ref[...] + b2_ref[...]
    h2_ref[...] = jnp.maximum(h, 0.0)


def _tc3(s2p, y2, dinv, b2):
    grid = N // _R1
    return pl.pallas_call(
        _tc3_body,
        grid=(grid,),
        in_specs=[
            pl.BlockSpec((NC, _R1, 16), lambda i: (0, i, 0)),
            pl.BlockSpec((_R1, 16), lambda i: (i, 0)),
            pl.BlockSpec((_R1, 1), lambda i: (i, 0)),
            pl.BlockSpec((1, 16), lambda i: (0, 0)),
        ],
        out_specs=pl.BlockSpec((_R1, 16), lambda i: (i, 0)),
        out_shape=jax.ShapeDtypeStruct((N, 16), jnp.float32),
    )(s2p, y2, dinv, b2)


def _tc4_body(h_ref, wfct_ref, bfc_ref, wfc2t_ref, bfc2_ref, out_ref):
    z = jnp.dot(h_ref[...], wfct_ref[...], preferred_element_type=jnp.float32)
    z = jnp.maximum(z + bfc_ref[...], 0.0)
    o = jnp.dot(z, wfc2t_ref[...], preferred_element_type=jnp.float32)
    out_ref[...] = jax.nn.sigmoid(o + bfc2_ref[...])


def _tc4(hflat, WfcT, bfc, Wfc2T, bfc2):
    return pl.pallas_call(
        _tc4_body,
        out_shape=jax.ShapeDtypeStruct((50, 1), jnp.float32),
    )(hflat, WfcT, bfc, Wfc2T, bfc2)


def kernel(x, edge_index, edge_attr, W1, b1, W2, b2, Wfc, bfc, Wfc2, bfc2):
    row = edge_index[0]
    col = edge_index[1]
    ew = edge_attr
    ew_bits = lax.bitcast_convert_type(ew, jnp.int32)
    packed = jnp.stack([row, col, ew_bits], 0)
    nch = EPW // 400
    edata = (packed.reshape(3, NW, nch, 400)
             .transpose(1, 2, 0, 3)
             .reshape(NW * nch, 3, 400))

    degp = _deg_kernel(edata)
    xw1 = _tcmm(x.T, W1)
    y1, dinv = _tcpre(xw1, degp.reshape(NC, NPAD).T)
    s1p = _msg32(edata, y1)
    y2 = _tc2(s1p, y1, dinv, W2, b1.reshape(1, 32))
    s2p = _msg16(edata, y2)
    h2 = _tc3(s2p, y2, dinv, b2.reshape(1, 16))
    hflat = h2.reshape(50, 16000)
    return _tc4(hflat, Wfc.T, bfc.reshape(1, 64), Wfc2.T, bfc2.reshape(1, 1))
